# Initial kernel scaffold; baseline (speedup 1.0000x reference)
import jax
import jax.numpy as jnp
from jax.experimental import pallas as pl


def kernel(node_embeddings, hyperedge_indices):
    raise NotImplementedError("write your pallas kernel here")


# speedup vs baseline: 19.8022x; 19.8022x over previous
import functools

import jax
import jax.numpy as jnp
from jax import lax
from jax.experimental import pallas as pl
from jax.experimental.pallas import tpu as pltpu
from jax.experimental.pallas import tpu_sc as plsc

NODES = 100000
FEAT = 128
N_IDX = 500000
NC, NS = 2, 16
NW = NC * NS
PER_W = N_IDX // NW
CH = 125
NCH = PER_W // CH

@functools.cache
def _get_histogram():
    mesh = plsc.VectorSubcoreMesh(
        core_axis_name="c", subcore_axis_name="s", num_cores=NC, num_subcores=NS
    )
    return pl.kernel(
        _histogram_body,
        out_type=jax.ShapeDtypeStruct((NC, NODES), jnp.float32),
        mesh=mesh,
        scratch_types=[
            pltpu.VMEM((NCH, CH), jnp.int32),
            pltpu.VMEM((128,), jnp.float32),
            pltpu.VMEM_SHARED((NODES,), jnp.float32),
        ],
    )


def _histogram_body(idx_hbm, zeros_hbm, out_hbm, idx_v, ones_v, hist_sh):
    c = lax.axis_index("c")
    s = lax.axis_index("s")
    w = c * NS + s

    for i in range(8):
        ones_v[pl.ds(i * 16, 16)] = jnp.ones((16,), jnp.float32)

    pltpu.sync_copy(idx_hbm.at[w], idx_v)

    @pl.when(s == 0)
    def _():
        pltpu.sync_copy(zeros_hbm, hist_sh)

    plsc.subcore_barrier()

    @pl.loop(0, NCH)
    def _(j):
        pltpu.sync_copy(
            ones_v.at[pl.ds(0, CH)], hist_sh.at[idx_v.at[j]], add=True
        )

    plsc.subcore_barrier()

    @pl.when(s == 0)
    def _():
        pltpu.sync_copy(hist_sh, out_hbm.at[c])


ROWS_BLK = 10000
GRID = NODES // ROWS_BLK


def _matvec_body(counts_ref, table_ref, out_ref):
    i = pl.program_id(0)
    w = jnp.sum(counts_ref[...].reshape(NC, ROWS_BLK), axis=0, keepdims=True)
    part = lax.dot_general(
        w, table_ref[...], (((1,), (0,)), ((), ())),
        preferred_element_type=jnp.float32,
    )

    @pl.when(i == 0)
    def _():
        out_ref[...] = part

    @pl.when(i != 0)
    def _():
        out_ref[...] += part

    @pl.when(i == GRID - 1)
    def _():
        out_ref[...] *= jnp.float32(1.0 / N_IDX)


_matvec = pl.pallas_call(
    _matvec_body,
    grid=(GRID,),
    in_specs=[
        pl.BlockSpec((NC, 1, 1, ROWS_BLK), lambda i: (0, i, 0, 0)),
        pl.BlockSpec((ROWS_BLK, FEAT), lambda i: (i, 0)),
    ],
    out_specs=pl.BlockSpec((1, FEAT), lambda i: (0, 0)),
    out_shape=jax.ShapeDtypeStruct((1, FEAT), jnp.float32),
)


def kernel(node_embeddings, hyperedge_indices):
    idx = hyperedge_indices.astype(jnp.int32).reshape(NW, NCH, CH)
    zeros = jnp.zeros((NODES,), jnp.float32)
    hist = _get_histogram()(idx, zeros)
    counts = hist.reshape(NC, GRID, 1, ROWS_BLK)
    out = _matvec(counts, node_embeddings)
    return out[0]


# speedup vs baseline: 22.9448x; 1.1587x over previous
import functools

import jax
import jax.numpy as jnp
from jax import lax
from jax.experimental import pallas as pl
from jax.experimental.pallas import tpu as pltpu
from jax.experimental.pallas import tpu_sc as plsc

NODES = 100000
FEAT = 128
N_IDX = 500000
NC, NS = 2, 16
NW = NC * NS
PER_W = N_IDX // NW
CH = 125
NCH = PER_W // CH

@functools.cache
def _get_histogram():
    mesh = plsc.VectorSubcoreMesh(
        core_axis_name="c", subcore_axis_name="s", num_cores=NC, num_subcores=NS
    )
    return pl.kernel(
        _histogram_body,
        out_type=jax.ShapeDtypeStruct((NC, NODES), jnp.float32),
        mesh=mesh,
        scratch_types=[
            pltpu.VMEM((NCH, CH), jnp.int32),
            pltpu.VMEM((128,), jnp.float32),
            pltpu.VMEM_SHARED((NODES,), jnp.float32),
            pltpu.SemaphoreType.DMA,
        ],
    )


_SCATTER_DEPTH = 8


def _histogram_body(idx_hbm, zeros_hbm, out_hbm, idx_v, ones_v, hist_sh, sem):
    c = lax.axis_index("c")
    s = lax.axis_index("s")
    w = c * NS + s

    for i in range(8):
        ones_v[pl.ds(i * 16, 16)] = jnp.ones((16,), jnp.float32)

    pltpu.sync_copy(idx_hbm.at[w], idx_v)

    @pl.when(s == 0)
    def _():
        pltpu.sync_copy(zeros_hbm, hist_sh)

    plsc.subcore_barrier()

    src = ones_v.at[pl.ds(0, CH)]
    head = pltpu.async_copy(src, hist_sh.at[idx_v.at[0]], sem, add=True)
    for j0 in range(1, _SCATTER_DEPTH):
        pltpu.async_copy(src, hist_sh.at[idx_v.at[j0]], sem, add=True)

    @pl.loop(_SCATTER_DEPTH, NCH)
    def _(j):
        head.wait()
        pltpu.async_copy(src, hist_sh.at[idx_v.at[j]], sem, add=True)

    for _ in range(_SCATTER_DEPTH):
        head.wait()

    plsc.subcore_barrier()

    @pl.when(s == 0)
    def _():
        pltpu.sync_copy(hist_sh, out_hbm.at[c])


ROWS_BLK = 10000
GRID = NODES // ROWS_BLK


def _matvec_body(counts_ref, table_ref, out_ref):
    i = pl.program_id(0)
    w = jnp.sum(counts_ref[...].reshape(NC, ROWS_BLK), axis=0, keepdims=True)
    part = lax.dot_general(
        w, table_ref[...], (((1,), (0,)), ((), ())),
        preferred_element_type=jnp.float32,
    )

    @pl.when(i == 0)
    def _():
        out_ref[...] = part

    @pl.when(i != 0)
    def _():
        out_ref[...] += part

    @pl.when(i == GRID - 1)
    def _():
        out_ref[...] *= jnp.float32(1.0 / N_IDX)


_matvec = pl.pallas_call(
    _matvec_body,
    grid=(GRID,),
    in_specs=[
        pl.BlockSpec((NC, 1, 1, ROWS_BLK), lambda i: (0, i, 0, 0)),
        pl.BlockSpec((ROWS_BLK, FEAT), lambda i: (i, 0)),
    ],
    out_specs=pl.BlockSpec((1, FEAT), lambda i: (0, 0)),
    out_shape=jax.ShapeDtypeStruct((1, FEAT), jnp.float32),
)


def kernel(node_embeddings, hyperedge_indices):
    idx = hyperedge_indices.astype(jnp.int32).reshape(NW, NCH, CH)
    zeros = jnp.zeros((NODES,), jnp.float32)
    hist = _get_histogram()(idx, zeros)
    counts = hist.reshape(NC, GRID, 1, ROWS_BLK)
    out = _matvec(counts, node_embeddings)
    return out[0]
